# baseline (speedup 1.0000x reference)
import functools

import jax
import jax.numpy as jnp
from jax import lax
from jax.experimental import pallas as pl
from jax.experimental.pallas import tpu as pltpu
from jax.experimental.pallas import tpu_sc as plsc

L = 16


def _build(B, S, H, NC, NS):
    NW = NC * NS
    assert S % NW == 0 and H % L == 0
    BAND = S // NW
    NB = B
    U = 8
    assert BAND == L and NB % 2 == 0 and H % U == 0

    mesh = plsc.VectorSubcoreMesh(core_axis_name="c", subcore_axis_name="s")

    @functools.partial(
        pl.kernel,
        mesh=mesh,
        out_type=jax.ShapeDtypeStruct((B * S, H), jnp.float32),
        compiler_params=pltpu.CompilerParams(
            use_tc_tiling_on_sc=False, needs_layout_passes=False),
        scratch_types=[
            pltpu.VMEM((H, L), jnp.float32),
            pltpu.VMEM((H, L), jnp.float32),
            pltpu.VMEM((H, L), jnp.float32),
            pltpu.VMEM((H, L), jnp.float32),
            pltpu.VMEM((BAND,), jnp.int32),
            pltpu.VMEM((BAND,), jnp.int32),
            pltpu.VMEM((BAND, H), jnp.float32),
            pltpu.VMEM((BAND, H), jnp.float32),
            pltpu.VMEM((BAND, H), jnp.float32),
            pltpu.VMEM((BAND, H), jnp.float32),
            pltpu.SemaphoreType.DMA,
            pltpu.SemaphoreType.DMA,
            pltpu.SemaphoreType.DMA,
            pltpu.SemaphoreType.DMA,
        ],
    )
    def emb_ln(ids_hbm, table_hbm, posr_hbm, gsp_hbm, bsp_hbm, out_hbm,
               pos_t, gsp_v, bsp_v, emb_t, idx0, idx1, tok0, tok1,
               out0, out1, gsem0, gsem1, osem0, osem1):
        c = lax.axis_index("c")
        s = lax.axis_index("s")
        wid = s * NC + c
        s0 = wid * BAND

        pltpu.sync_copy(posr_hbm.at[wid], pos_t)
        pltpu.sync_copy(gsp_hbm, gsp_v)
        pltpu.sync_copy(bsp_hbm, bsp_v)

        idx = (idx0, idx1)
        tok = (tok0, tok1)
        outb = (out0, out1)
        gsem = (gsem0, gsem1)
        osem = (osem0, osem1)

        lanes = lax.iota(jnp.int32, L)

        def start_gather(g, p):
            base = g * S + s0
            pltpu.sync_copy(ids_hbm.at[pl.ds(base, BAND)], idx[p])
            pltpu.async_copy(table_hbm.at[idx[p]], tok[p], gsem[p])

        def compute(p):
            zero = jnp.zeros((L,), jnp.float32)
            colv0 = jnp.zeros((L,), jnp.int32)

            def body1(i, carry):
                a0, a1, a2, a3, q0, q1, q2, q3, colv = carry
                accs = [a0, a1, a2, a3]
                sqs = [q0, q1, q2, q3]
                for u in range(U):
                    k = i * U + u
                    tokc = plsc.load_gather(tok[p], [lanes, colv])
                    x = tokc + pos_t[k, :]
                    emb_t[k, :] = x
                    accs[u % 4] = accs[u % 4] + x
                    sqs[u % 4] = sqs[u % 4] + x * x
                    colv = colv + 1
                return (*accs, *sqs, colv)

            a0, a1, a2, a3, q0, q1, q2, q3, _ = lax.fori_loop(
                0, H // U, body1,
                (zero, zero, zero, zero, zero, zero, zero, zero, colv0))

            mean_v = ((a0 + a1) + (a2 + a3)) * (1.0 / H)
            var_v = ((q0 + q1) + (q2 + q3)) * (1.0 / H) - mean_v * mean_v
            var_v = jnp.maximum(var_v, 0.0) + 1e-12
            iv = lax.bitcast_convert_type(var_v, jnp.int32)
            yi = jnp.full((L,), 0x5F3759DF, jnp.int32) - lax.shift_right_logical(iv, 1)
            y = lax.bitcast_convert_type(yi, jnp.float32)
            vh = var_v * 0.5
            for _ in range(3):
                y = y * (1.5 - vh * y * y)
            inv_v = y
            mm_v = mean_v * inv_v

            def body2(i, colv):
                for u in range(U):
                    k = i * U + u
                    x = emb_t[k, :]
                    yv = x * inv_v - mm_v
                    yv = yv * gsp_v[k, :] + bsp_v[k, :]
                    plsc.store_scatter(outb[p], [lanes, colv], yv)
                    colv = colv + 1
                return colv

            lax.fori_loop(0, H // U, body2, colv0)

        def chunk(g, p):
            pltpu.make_async_copy(table_hbm.at[idx[p]], tok[p], gsem[p]).wait()

            @pl.when(g >= 2)
            def _():
                pltpu.make_async_copy(
                    outb[p], out_hbm.at[pl.ds(0, BAND)], osem[p]).wait()

            compute(p)

            base = g * S + s0
            pltpu.async_copy(outb[p], out_hbm.at[pl.ds(base, BAND)], osem[p])

            @pl.when(g + 2 < NB)
            def _():
                start_gather(g + 2, p)

        start_gather(0, 0)
        start_gather(1, 1)

        def outer(gg, carry):
            chunk(gg * 2, 0)
            chunk(gg * 2 + 1, 1)
            return carry
        lax.fori_loop(0, NB // 2, outer, 0)

        pltpu.make_async_copy(outb[0], out_hbm.at[pl.ds(0, BAND)], osem[0]).wait()
        pltpu.make_async_copy(outb[1], out_hbm.at[pl.ds(0, BAND)], osem[1]).wait()

    return emb_ln


def kernel(ipt_ids, token_table, pos_table, gamma, beta):
    B, S = ipt_ids.shape
    H = token_table.shape[1]
    info = plsc.get_sparse_core_info()
    NC, NS = info.num_cores, info.num_subcores
    NW = NC * NS
    BAND = S // NW
    ids = ipt_ids.reshape(B * S).astype(jnp.int32)
    pos_r = pos_table.reshape(NW, BAND, H).transpose(0, 2, 1)
    gsp = jnp.broadcast_to(gamma[:, None], (H, L))
    bsp = jnp.broadcast_to(beta[:, None], (H, L))
    fn = _build(B, S, H, NC, NS)
    out = fn(ids, token_table, pos_r, gsp, bsp)
    return out.reshape(B, S, H)


# speedup vs baseline: 1.4020x; 1.4020x over previous
import functools

import jax
import jax.numpy as jnp
from jax import lax
from jax.experimental import pallas as pl
from jax.experimental.pallas import tpu as pltpu
from jax.experimental.pallas import tpu_sc as plsc

L = 16


def _build(B, S, H, NC, NS):
    NW = NC * NS
    assert S % NW == 0 and H % L == 0
    BAND = S // NW
    NB = B
    U = 8
    assert BAND == L and NB % 2 == 0 and H % U == 0

    mesh = plsc.VectorSubcoreMesh(core_axis_name="c", subcore_axis_name="s")

    @functools.partial(
        pl.kernel,
        mesh=mesh,
        out_type=jax.ShapeDtypeStruct((B * S, H), jnp.float32),
        compiler_params=pltpu.CompilerParams(
            use_tc_tiling_on_sc=False, needs_layout_passes=False),
        scratch_types=[
            pltpu.VMEM((H, L), jnp.float32),
            pltpu.VMEM((H, L), jnp.float32),
            pltpu.VMEM((H, L), jnp.float32),
            pltpu.VMEM((H, L), jnp.float32),
            pltpu.VMEM((BAND,), jnp.int32),
            pltpu.VMEM((BAND,), jnp.int32),
            pltpu.VMEM((BAND, H), jnp.float32),
            pltpu.VMEM((BAND, H), jnp.float32),
            pltpu.VMEM((BAND, H), jnp.float32),
            pltpu.VMEM((BAND, H), jnp.float32),
            pltpu.SemaphoreType.DMA,
            pltpu.SemaphoreType.DMA,
            pltpu.SemaphoreType.DMA,
            pltpu.SemaphoreType.DMA,
        ],
    )
    def emb_ln(ids_hbm, table_hbm, posr_hbm, gsp_hbm, bsp_hbm, out_hbm,
               pos_t, gsp_v, bsp_v, emb_t, idx0, idx1, tok0, tok1,
               out0, out1, gsem0, gsem1, osem0, osem1):
        c = lax.axis_index("c")
        s = lax.axis_index("s")
        wid = s * NC + c
        s0 = wid * BAND

        pltpu.sync_copy(posr_hbm.at[wid], pos_t)
        pltpu.sync_copy(gsp_hbm, gsp_v)
        pltpu.sync_copy(bsp_hbm, bsp_v)

        idx = (idx0, idx1)
        tok = (tok0, tok1)
        outb = (out0, out1)
        gsem = (gsem0, gsem1)
        osem = (osem0, osem1)

        lanes = lax.iota(jnp.int32, L)

        def start_gather(g, p):
            base = g * S + s0
            pltpu.sync_copy(ids_hbm.at[pl.ds(base, BAND)], idx[p])
            pltpu.async_copy(table_hbm.at[idx[p]], tok[p], gsem[p])

        def compute(p):
            zero = jnp.zeros((L,), jnp.float32)
            NMAIN = H - L
            assert NMAIN % U == 0

            def body1(i, carry):
                a0, a1, a2, a3, q0, q1, q2, q3, colv = carry
                accs = [a0, a1, a2, a3]
                sqs = [q0, q1, q2, q3]
                for u in range(U):
                    k = i * U + u
                    tokc = plsc.load_gather(tok[p], [lanes, colv])
                    x = tokc + pos_t[k, :]
                    emb_t[k, :] = x
                    accs[u % 4] = accs[u % 4] + x
                    sqs[u % 4] = sqs[u % 4] + x * x
                    colv = colv + 1
                return (*accs, *sqs, colv)

            a0, a1, a2, a3, q0, q1, q2, q3, _ = lax.fori_loop(
                0, NMAIN // U, body1,
                (zero, zero, zero, zero, zero, zero, zero, zero, lanes))

            accs = [a0, a1, a2, a3]
            sqs = [q0, q1, q2, q3]
            for u in range(L):
                k = NMAIN + u
                colv = jnp.full((L,), k, jnp.int32) + lanes
                colv = jnp.where(colv >= H, colv - H, colv)
                tokc = plsc.load_gather(tok[p], [lanes, colv])
                x = tokc + pos_t[k, :]
                emb_t[k, :] = x
                accs[u % 4] = accs[u % 4] + x
                sqs[u % 4] = sqs[u % 4] + x * x
            a0, a1, a2, a3 = accs
            q0, q1, q2, q3 = sqs

            mean_v = ((a0 + a1) + (a2 + a3)) * (1.0 / H)
            var_v = ((q0 + q1) + (q2 + q3)) * (1.0 / H) - mean_v * mean_v
            var_v = jnp.maximum(var_v, 0.0) + 1e-12
            iv = lax.bitcast_convert_type(var_v, jnp.int32)
            yi = jnp.full((L,), 0x5F3759DF, jnp.int32) - lax.shift_right_logical(iv, 1)
            y = lax.bitcast_convert_type(yi, jnp.float32)
            vh = var_v * 0.5
            for _ in range(3):
                y = y * (1.5 - vh * y * y)
            inv_v = y
            mm_v = mean_v * inv_v

            def body2(i, colv):
                for u in range(U):
                    k = i * U + u
                    x = emb_t[k, :]
                    yv = x * inv_v - mm_v
                    yv = yv * gsp_v[k, :] + bsp_v[k, :]
                    plsc.store_scatter(outb[p], [lanes, colv], yv)
                    colv = colv + 1
                return colv

            lax.fori_loop(0, NMAIN // U, body2, lanes)
            for u in range(L):
                k = NMAIN + u
                colv = jnp.full((L,), k, jnp.int32) + lanes
                colv = jnp.where(colv >= H, colv - H, colv)
                x = emb_t[k, :]
                yv = x * inv_v - mm_v
                yv = yv * gsp_v[k, :] + bsp_v[k, :]
                plsc.store_scatter(outb[p], [lanes, colv], yv)

        def chunk(g, p):
            pltpu.make_async_copy(table_hbm.at[idx[p]], tok[p], gsem[p]).wait()

            @pl.when(g >= 2)
            def _():
                pltpu.make_async_copy(
                    outb[p], out_hbm.at[pl.ds(0, BAND)], osem[p]).wait()

            compute(p)

            base = g * S + s0
            pltpu.async_copy(outb[p], out_hbm.at[pl.ds(base, BAND)], osem[p])

            @pl.when(g + 2 < NB)
            def _():
                start_gather(g + 2, p)

        start_gather(0, 0)
        start_gather(1, 1)

        def outer(gg, carry):
            chunk(gg * 2, 0)
            chunk(gg * 2 + 1, 1)
            return carry
        lax.fori_loop(0, NB // 2, outer, 0)

        pltpu.make_async_copy(outb[0], out_hbm.at[pl.ds(0, BAND)], osem[0]).wait()
        pltpu.make_async_copy(outb[1], out_hbm.at[pl.ds(0, BAND)], osem[1]).wait()

    return emb_ln


def kernel(ipt_ids, token_table, pos_table, gamma, beta):
    B, S = ipt_ids.shape
    H = token_table.shape[1]
    info = plsc.get_sparse_core_info()
    NC, NS = info.num_cores, info.num_subcores
    NW = NC * NS
    BAND = S // NW
    ids = ipt_ids.reshape(B * S).astype(jnp.int32)
    feat = (jnp.arange(H)[:, None] + jnp.arange(BAND)[None, :]) % H
    posb = pos_table.reshape(NW, BAND, H)
    pos_r = posb[:, jnp.arange(BAND)[None, :], feat]
    gsp = gamma[feat]
    bsp = beta[feat]
    fn = _build(B, S, H, NC, NS)
    out = fn(ids, token_table, pos_r, gsp, bsp)
    return out.reshape(B, S, H)


# speedup vs baseline: 6.9622x; 4.9659x over previous
import functools

import jax
import jax.numpy as jnp
from jax import lax
from jax.experimental import pallas as pl
from jax.experimental.pallas import tpu as pltpu
from jax.experimental.pallas import tpu_sc as plsc

GCH = 64


def _build_gather(nrows, V, H, NC, NS):
    NW = NC * NS
    assert nrows % (NW * GCH) == 0
    per_w = nrows // NW
    nch = per_w // GCH
    assert nch >= 2

    mesh = plsc.VectorSubcoreMesh(core_axis_name="c", subcore_axis_name="s")

    @functools.partial(
        pl.kernel,
        mesh=mesh,
        out_type=jax.ShapeDtypeStruct((nrows, H), jnp.float32),
        scratch_types=[
            pltpu.VMEM((per_w,), jnp.int32),
            pltpu.VMEM((GCH, H), jnp.float32),
            pltpu.VMEM((GCH, H), jnp.float32),
            pltpu.SemaphoreType.DMA,
            pltpu.SemaphoreType.DMA,
            pltpu.SemaphoreType.DMA,
            pltpu.SemaphoreType.DMA,
        ],
    )
    def gather_rows(ids_hbm, table_hbm, out_hbm, idx_v, tok0, tok1,
                    gsem0, gsem1, osem0, osem1):
        c = lax.axis_index("c")
        s = lax.axis_index("s")
        wid = s * NC + c
        base = wid * per_w
        pltpu.sync_copy(ids_hbm.at[pl.ds(base, per_w)], idx_v)
        tok = (tok0, tok1)
        gsem = (gsem0, gsem1)
        osem = (osem0, osem1)

        def start_gather(j):
            p = j % 2
            pltpu.async_copy(
                table_hbm.at[idx_v.at[pl.ds(j * GCH, GCH)]], tok[p], gsem[p])

        start_gather(0)
        start_gather(1)
        for j in range(nch):
            p = j % 2
            pltpu.make_async_copy(
                table_hbm.at[idx_v.at[pl.ds(j * GCH, GCH)]], tok[p],
                gsem[p]).wait()
            pltpu.async_copy(
                tok[p], out_hbm.at[pl.ds(base + j * GCH, GCH)], osem[p])
            if j + 2 < nch:
                pltpu.make_async_copy(
                    tok[p], out_hbm.at[pl.ds(base + j * GCH, GCH)],
                    osem[p]).wait()
                start_gather(j + 2)
        for j in (nch - 2, nch - 1):
            p = j % 2
            pltpu.make_async_copy(
                tok[p], out_hbm.at[pl.ds(base + j * GCH, GCH)],
                osem[p]).wait()

    return gather_rows


def _ln_body(emb_ref, pos_ref, gamma_ref, beta_ref, out_ref):
    x = emb_ref[...] + pos_ref[...]
    mean = jnp.mean(x, axis=-1, keepdims=True)
    xc = x - mean
    var = jnp.mean(xc * xc, axis=-1, keepdims=True)
    y = xc * jax.lax.rsqrt(var + 1e-12)
    out_ref[...] = (y * gamma_ref[...] + beta_ref[...])[None]


def _build_ln(B, S, H, RB):
    nsb = S // RB

    return pl.pallas_call(
        _ln_body,
        grid=(B, nsb),
        in_specs=[
            pl.BlockSpec((RB, H), lambda b, j: (b * nsb + j, 0)),
            pl.BlockSpec((RB, H), lambda b, j: (j, 0)),
            pl.BlockSpec((1, H), lambda b, j: (0, 0)),
            pl.BlockSpec((1, H), lambda b, j: (0, 0)),
        ],
        out_specs=pl.BlockSpec((1, RB, H), lambda b, j: (b, j, 0)),
        out_shape=jax.ShapeDtypeStruct((B, S, H), jnp.float32),
        compiler_params=pltpu.CompilerParams(
            dimension_semantics=("parallel", "arbitrary"),
        ),
    )


def kernel(ipt_ids, token_table, pos_table, gamma, beta):
    B, S = ipt_ids.shape
    V, H = token_table.shape
    info = plsc.get_sparse_core_info()
    NC, NS = info.num_cores, info.num_subcores
    ids = ipt_ids.reshape(B * S).astype(jnp.int32)
    emb = _build_gather(B * S, V, H, NC, NS)(ids, token_table)
    RB = 256
    out = _build_ln(B, S, H, RB)(
        emb, pos_table, gamma.reshape(1, H), beta.reshape(1, H))
    return out


# speedup vs baseline: 7.7489x; 1.1130x over previous
import functools

import jax
import jax.numpy as jnp
from jax import lax
from jax.experimental import pallas as pl
from jax.experimental.pallas import tpu as pltpu
from jax.experimental.pallas import tpu_sc as plsc

GCH = 64


def _build_gather(nrows, V, H, NC, NS):
    NW = NC * NS
    assert nrows % (NW * GCH) == 0
    per_w = nrows // NW
    nch = per_w // GCH
    assert nch >= 2

    mesh = plsc.VectorSubcoreMesh(core_axis_name="c", subcore_axis_name="s")

    @functools.partial(
        pl.kernel,
        mesh=mesh,
        out_type=jax.ShapeDtypeStruct((nrows, H), jnp.float32),
        scratch_types=[
            pltpu.VMEM((per_w,), jnp.int32),
            pltpu.VMEM((GCH, H), jnp.float32),
            pltpu.VMEM((GCH, H), jnp.float32),
            pltpu.SemaphoreType.DMA,
            pltpu.SemaphoreType.DMA,
            pltpu.SemaphoreType.DMA,
            pltpu.SemaphoreType.DMA,
        ],
    )
    def gather_rows(ids_hbm, table_hbm, out_hbm, idx_v, tok0, tok1,
                    gsem0, gsem1, osem0, osem1):
        c = lax.axis_index("c")
        s = lax.axis_index("s")
        wid = s * NC + c
        base = wid * per_w
        pltpu.sync_copy(ids_hbm.at[pl.ds(base, per_w)], idx_v)
        tok = (tok0, tok1)
        gsem = (gsem0, gsem1)
        osem = (osem0, osem1)

        def start_gather(j):
            p = j % 2
            pltpu.async_copy(
                table_hbm.at[idx_v.at[pl.ds(j * GCH, GCH)]], tok[p], gsem[p])

        start_gather(0)
        start_gather(1)
        for j in range(nch):
            p = j % 2
            pltpu.make_async_copy(
                table_hbm.at[idx_v.at[pl.ds(j * GCH, GCH)]], tok[p],
                gsem[p]).wait()
            pltpu.async_copy(
                tok[p], out_hbm.at[pl.ds(base + j * GCH, GCH)], osem[p])
            if j + 2 < nch:
                pltpu.make_async_copy(
                    tok[p], out_hbm.at[pl.ds(base + j * GCH, GCH)],
                    osem[p]).wait()
                start_gather(j + 2)
        for j in (nch - 2, nch - 1):
            p = j % 2
            pltpu.make_async_copy(
                tok[p], out_hbm.at[pl.ds(base + j * GCH, GCH)],
                osem[p]).wait()

    return gather_rows


def _ln_body(emb_ref, pos_ref, gamma_ref, beta_ref, out_ref):
    x = emb_ref[...] + pos_ref[...]
    mean = jnp.mean(x, axis=-1, keepdims=True)
    xc = x - mean
    var = jnp.mean(xc * xc, axis=-1, keepdims=True)
    y = xc * jax.lax.rsqrt(var + 1e-12)
    out_ref[...] = (y * gamma_ref[...] + beta_ref[...])[None]


def _build_ln(B, S, H, RB):
    nsb = S // RB

    return pl.pallas_call(
        _ln_body,
        grid=(nsb, B),
        in_specs=[
            pl.BlockSpec((RB, H), lambda j, b: (b * nsb + j, 0)),
            pl.BlockSpec((RB, H), lambda j, b: (j, 0)),
            pl.BlockSpec((1, H), lambda j, b: (0, 0)),
            pl.BlockSpec((1, H), lambda j, b: (0, 0)),
        ],
        out_specs=pl.BlockSpec((1, RB, H), lambda j, b: (b, j, 0)),
        out_shape=jax.ShapeDtypeStruct((B, S, H), jnp.float32),
        compiler_params=pltpu.CompilerParams(
            dimension_semantics=("parallel", "parallel"),
        ),
    )


def kernel(ipt_ids, token_table, pos_table, gamma, beta):
    B, S = ipt_ids.shape
    V, H = token_table.shape
    info = plsc.get_sparse_core_info()
    NC, NS = info.num_cores, info.num_subcores
    ids = ipt_ids.reshape(B * S).astype(jnp.int32)
    emb = _build_gather(B * S, V, H, NC, NS)(ids, token_table)
    RB = 256
    out = _build_ln(B, S, H, RB)(
        emb, pos_table, gamma.reshape(1, H), beta.reshape(1, H))
    return out


# speedup vs baseline: 9.0728x; 1.1709x over previous
import functools

import jax
import jax.numpy as jnp
from jax import lax
from jax.experimental import pallas as pl
from jax.experimental.pallas import tpu as pltpu
from jax.experimental.pallas import tpu_sc as plsc

GCH = 64


def _build_gather(nrows, V, H, NC, NS):
    NW = NC * NS
    assert nrows % (NW * GCH) == 0
    per_w = nrows // NW
    nch = per_w // GCH
    assert nch >= 2

    mesh = plsc.VectorSubcoreMesh(core_axis_name="c", subcore_axis_name="s")

    @functools.partial(
        pl.kernel,
        mesh=mesh,
        out_type=jax.ShapeDtypeStruct((nrows, H), jnp.float32),
        scratch_types=[
            pltpu.VMEM((per_w,), jnp.int32),
            pltpu.VMEM((GCH, H), jnp.float32),
            pltpu.VMEM((GCH, H), jnp.float32),
            pltpu.SemaphoreType.DMA,
            pltpu.SemaphoreType.DMA,
            pltpu.SemaphoreType.DMA,
            pltpu.SemaphoreType.DMA,
        ],
    )
    def gather_rows(ids_hbm, table_hbm, out_hbm, idx_v, tok0, tok1,
                    gsem0, gsem1, osem0, osem1):
        c = lax.axis_index("c")
        s = lax.axis_index("s")
        wid = s * NC + c
        base = wid * per_w
        pltpu.sync_copy(ids_hbm.at[pl.ds(base, per_w)], idx_v)
        tok = (tok0, tok1)
        gsem = (gsem0, gsem1)
        osem = (osem0, osem1)

        def start_gather(j):
            p = j % 2
            pltpu.async_copy(
                table_hbm.at[idx_v.at[pl.ds(j * GCH, GCH)]], tok[p], gsem[p])

        start_gather(0)
        start_gather(1)
        for j in range(nch):
            p = j % 2
            pltpu.make_async_copy(
                table_hbm.at[idx_v.at[pl.ds(j * GCH, GCH)]], tok[p],
                gsem[p]).wait()
            pltpu.async_copy(
                tok[p], out_hbm.at[pl.ds(base + j * GCH, GCH)], osem[p])
            if j + 2 < nch:
                pltpu.make_async_copy(
                    tok[p], out_hbm.at[pl.ds(base + j * GCH, GCH)],
                    osem[p]).wait()
                start_gather(j + 2)
        for j in (nch - 2, nch - 1):
            p = j % 2
            pltpu.make_async_copy(
                tok[p], out_hbm.at[pl.ds(base + j * GCH, GCH)],
                osem[p]).wait()

    return gather_rows


def _ln_body(emb_ref, pos_ref, gamma_ref, beta_ref, out_ref):
    x = emb_ref[...] + pos_ref[...]
    mean = jnp.mean(x, axis=-1, keepdims=True)
    xc = x - mean
    var = jnp.mean(xc * xc, axis=-1, keepdims=True)
    y = xc * jax.lax.rsqrt(var + 1e-12)
    out_ref[...] = (y * gamma_ref[...] + beta_ref[...])[None]


def _build_ln(B, S, H, RB):
    nsb = S // RB

    return pl.pallas_call(
        _ln_body,
        grid=(nsb, B),
        in_specs=[
            pl.BlockSpec((RB, H), lambda j, b: (b * nsb + j, 0)),
            pl.BlockSpec((RB, H), lambda j, b: (j, 0)),
            pl.BlockSpec((1, H), lambda j, b: (0, 0)),
            pl.BlockSpec((1, H), lambda j, b: (0, 0)),
        ],
        out_specs=pl.BlockSpec((1, RB, H), lambda j, b: (b, j, 0)),
        out_shape=jax.ShapeDtypeStruct((B, S, H), jnp.float32),
        compiler_params=pltpu.CompilerParams(
            dimension_semantics=("parallel", "parallel"),
        ),
    )


def kernel(ipt_ids, token_table, pos_table, gamma, beta):
    B, S = ipt_ids.shape
    V, H = token_table.shape
    info = plsc.get_sparse_core_info()
    NC, NS = info.num_cores, info.num_subcores
    ids = ipt_ids.reshape(B * S).astype(jnp.int32)
    emb = _build_gather(B * S, V, H, NC, NS)(ids, token_table)
    RB = 512
    out = _build_ln(B, S, H, RB)(
        emb, pos_table, gamma.reshape(1, H), beta.reshape(1, H))
    return out


# speedup vs baseline: 9.8839x; 1.0894x over previous
import functools

import jax
import jax.numpy as jnp
from jax import lax
from jax.experimental import pallas as pl
from jax.experimental.pallas import tpu as pltpu
from jax.experimental.pallas import tpu_sc as plsc

GCH = 64


def _build_gather(nrows, V, H, NC, NS):
    NW = NC * NS
    assert nrows % (NW * GCH) == 0
    per_w = nrows // NW
    nch = per_w // GCH
    assert nch >= 2

    mesh = plsc.VectorSubcoreMesh(core_axis_name="c", subcore_axis_name="s")

    @functools.partial(
        pl.kernel,
        mesh=mesh,
        out_type=jax.ShapeDtypeStruct((nrows, H), jnp.float32),
        scratch_types=[
            pltpu.VMEM((per_w,), jnp.int32),
            pltpu.VMEM((GCH, H), jnp.float32),
            pltpu.VMEM((GCH, H), jnp.float32),
            pltpu.SemaphoreType.DMA,
            pltpu.SemaphoreType.DMA,
            pltpu.SemaphoreType.DMA,
            pltpu.SemaphoreType.DMA,
        ],
    )
    def gather_rows(ids_hbm, table_hbm, out_hbm, idx_v, tok0, tok1,
                    gsem0, gsem1, osem0, osem1):
        c = lax.axis_index("c")
        s = lax.axis_index("s")
        wid = s * NC + c
        base = wid * per_w
        pltpu.sync_copy(ids_hbm.at[pl.ds(base, per_w)], idx_v)
        tok = (tok0, tok1)
        gsem = (gsem0, gsem1)
        osem = (osem0, osem1)

        def start_gather(j):
            p = j % 2
            pltpu.async_copy(
                table_hbm.at[idx_v.at[pl.ds(j * GCH, GCH)]], tok[p], gsem[p])

        start_gather(0)
        start_gather(1)
        for j in range(nch):
            p = j % 2
            pltpu.make_async_copy(
                table_hbm.at[idx_v.at[pl.ds(j * GCH, GCH)]], tok[p],
                gsem[p]).wait()
            pltpu.async_copy(
                tok[p], out_hbm.at[pl.ds(base + j * GCH, GCH)], osem[p])
            if j + 2 < nch:
                pltpu.make_async_copy(
                    tok[p], out_hbm.at[pl.ds(base + j * GCH, GCH)],
                    osem[p]).wait()
                start_gather(j + 2)
        for j in (nch - 2, nch - 1):
            p = j % 2
            pltpu.make_async_copy(
                tok[p], out_hbm.at[pl.ds(base + j * GCH, GCH)],
                osem[p]).wait()

    return gather_rows


def _ln_body(emb_ref, pos_ref, gamma_ref, beta_ref, out_ref):
    x = emb_ref[...] + pos_ref[...]
    mean = jnp.mean(x, axis=-1, keepdims=True)
    xc = x - mean
    var = jnp.mean(xc * xc, axis=-1, keepdims=True)
    y = xc * jax.lax.rsqrt(var + 1e-12)
    out_ref[...] = y * gamma_ref[...] + beta_ref[...]


def _build_ln(N, H, RB):
    return pl.pallas_call(
        _ln_body,
        grid=(N // RB,),
        in_specs=[
            pl.BlockSpec((RB, H), lambda i: (i, 0)),
            pl.BlockSpec((RB, H), lambda i: (0, 0)),
            pl.BlockSpec((1, H), lambda i: (0, 0)),
            pl.BlockSpec((1, H), lambda i: (0, 0)),
        ],
        out_specs=pl.BlockSpec((RB, H), lambda i: (i, 0)),
        out_shape=jax.ShapeDtypeStruct((N, H), jnp.float32),
        compiler_params=pltpu.CompilerParams(
            dimension_semantics=("arbitrary",),
        ),
    )


def kernel(ipt_ids, token_table, pos_table, gamma, beta):
    B, S = ipt_ids.shape
    V, H = token_table.shape
    info = plsc.get_sparse_core_info()
    NC, NS = info.num_cores, info.num_subcores
    ids = ipt_ids.reshape(B * S).astype(jnp.int32)
    emb = _build_gather(B * S, V, H, NC, NS)(ids, token_table)
    RB = 1024
    pos_t = jnp.tile(pos_table, (RB // S, 1)) if RB > S else pos_table
    out = _build_ln(B * S, H, RB)(
        emb, pos_t, gamma.reshape(1, H), beta.reshape(1, H))
    return out.reshape(B, S, H)
